# Initial kernel scaffold; baseline (speedup 1.0000x reference)
import jax
import jax.numpy as jnp
from jax.experimental import pallas as pl


def kernel(x, edge_index, Wl1, bl1, Wr1, Wl2, bl2, Wr2, Wo, bo):
    raise NotImplementedError("write your pallas kernel here")


# speedup vs baseline: 2.9949x; 2.9949x over previous
import functools

import jax
import jax.numpy as jnp
from jax import lax
from jax.experimental import pallas as pl
from jax.experimental.pallas import tpu as pltpu
from jax.experimental.pallas import tpu_sc as plsc

N = 10000
E = 160000
DIN = 256
DH = 512

NCORES = 2
NTILES = 16
B = 128
DC = 128
RPT = 640
NP = NTILES * RPT
EPT = 10240
STEPS = EPT // B
NH = 2
HSTEPS = STEPS // NH
EP = NTILES * EPT


def _sc_segsum(tables, srcp, dstp, zrows, zdeg, ones1, with_deg):
    nt = len(tables)
    per_core = nt // NCORES
    mesh = plsc.VectorSubcoreMesh(core_axis_name="c", subcore_axis_name="s")
    out_type = [jax.ShapeDtypeStruct((NP, DC), jnp.float32) for _ in range(nt)]
    if with_deg:
        out_type.append(jax.ShapeDtypeStruct((NP,), jnp.float32))

    scratch = [
        pltpu.VMEM((HSTEPS, B), jnp.int32),
        pltpu.VMEM((HSTEPS, B), jnp.int32),
        pltpu.VMEM((B, DC), jnp.float32),
        pltpu.VMEM((B,), jnp.float32),
        pltpu.VMEM_SHARED((NP, DC), jnp.float32),
        pltpu.VMEM_SHARED((NP,), jnp.float32),
        pltpu.SemaphoreType.DMA,
    ]

    @functools.partial(pl.kernel, out_type=out_type, mesh=mesh,
                       scratch_types=scratch)
    def k(*refs):
        ins = refs[:nt + 5]
        tables_r = ins[:nt]
        srcp_r, dstp_r, zrows_r, zdeg_r, ones1_r = ins[nt:]
        outs = refs[nt + 5: nt + 5 + nt]
        if with_deg:
            deg_out = refs[nt + 5 + nt]
            rest = refs[nt + 6 + nt:]
        else:
            deg_out = None
            rest = refs[nt + 5 + nt:]
        src_v, dst_v, gbuf, ones_v, accum, degacc, sem = rest

        c = lax.axis_index("c")
        s = lax.axis_index("s")
        row0 = s * RPT

        pltpu.sync_copy(ones1_r, ones_v)

        for ch in range(per_core):
            pltpu.sync_copy(zrows_r, accum.at[pl.ds(row0, RPT)])
            if with_deg and ch == 0:
                @pl.when(c == 0)
                def _():
                    pltpu.sync_copy(zdeg_r, degacc.at[pl.ds(row0, RPT)])
            plsc.subcore_barrier()

            for h in range(NH):
                piece = s * NH + h
                pltpu.sync_copy(srcp_r.at[piece], src_v)
                pltpu.sync_copy(dstp_r.at[piece], dst_v)
                for cc in range(NCORES):
                    tab = tables_r[cc * per_core + ch]

                    @pl.when(c == cc)
                    def _(tab=tab):
                        def step(j, carry):
                            pltpu.async_copy(tab.at[src_v.at[j]], gbuf,
                                             sem).wait()
                            pltpu.sync_copy(gbuf, accum.at[dst_v.at[j]],
                                            add=True)
                            return carry
                        lax.fori_loop(0, HSTEPS, step, 0)

                if with_deg and ch == 0:
                    @pl.when(c == 0)
                    def _():
                        def dstep(j, carry):
                            pltpu.sync_copy(ones_v, degacc.at[dst_v.at[j]],
                                            add=True)
                            return carry
                        lax.fori_loop(0, HSTEPS, dstep, 0)

            plsc.subcore_barrier()

            for cc in range(NCORES):
                out = outs[cc * per_core + ch]

                @pl.when(c == cc)
                def _(out=out):
                    pltpu.sync_copy(accum.at[pl.ds(row0, RPT)],
                                    out.at[pl.ds(row0, RPT)])
            if with_deg and ch == 0:
                @pl.when(c == 0)
                def _():
                    pltpu.sync_copy(degacc.at[pl.ds(row0, RPT)],
                                    deg_out.at[pl.ds(row0, RPT)])

    args = list(tables) + [srcp, dstp, zrows, zdeg, ones1]
    return k(*args)


def _tc_layer1(aggs, deg1, xp, Wl1, bl1, Wr1):
    grid = (NP // RPT,)
    na = len(aggs)

    def body(*refs):
        ars = refs[:na]
        dg, xr, wl, bl, wr = refs[na:na + 5]
        outs = refs[na + 5:]
        rd = 1.0 / jnp.maximum(dg[...], 1.0)
        agg = jnp.concatenate([a[...] for a in ars], axis=1) * rd
        h = jnp.dot(agg, wl[...], preferred_element_type=jnp.float32)
        h = h + bl[...] + jnp.dot(xr[...], wr[...],
                                  preferred_element_type=jnp.float32)
        h = jax.nn.gelu(h)
        rows = (pl.program_id(0) * RPT
                + lax.broadcasted_iota(jnp.int32, (RPT, 1), 0))
        h = jnp.where(rows < N, h, 0.0)
        for i, o in enumerate(outs):
            o[...] = h[:, i * DC:(i + 1) * DC]

    blk = lambda r, cdim: pl.BlockSpec((r, cdim), lambda i: (i, 0))
    full = lambda shape: pl.BlockSpec(shape, lambda i: (0, 0))
    return pl.pallas_call(
        body,
        grid=grid,
        in_specs=[blk(RPT, DC)] * na
                 + [blk(RPT, 1), blk(RPT, DIN),
                    full((DIN, DH)), full((1, DH)), full((DIN, DH))],
        out_specs=[blk(RPT, DC)] * (DH // DC),
        out_shape=[jax.ShapeDtypeStruct((NP, DC), jnp.float32)] * (DH // DC),
    )(*aggs, deg1, xp, Wl1, bl1.reshape(1, DH), Wr1)


def _tc_layer2(aggs, deg1, ys, Wl2, bl2, Wr2, woT, bo):
    grid = (NP // RPT,)
    na = len(aggs)

    def body(*refs):
        ars = refs[:na]
        dg = refs[na]
        yrs = refs[na + 1:2 * na + 1]
        wl, bl, wr, wo, bor = refs[2 * na + 1:2 * na + 6]
        o = refs[2 * na + 6]
        rd = 1.0 / jnp.maximum(dg[...], 1.0)
        agg = jnp.concatenate([a[...] for a in ars], axis=1) * rd
        y = jnp.concatenate([yy[...] for yy in yrs], axis=1)
        acc = jnp.dot(agg, wl[...], preferred_element_type=jnp.float32)
        acc = acc + bl[...] + jnp.dot(y, wr[...],
                                      preferred_element_type=jnp.float32)
        x2 = jax.nn.gelu(acc)
        o[...] = jnp.sum(x2 * wo[...], axis=1, keepdims=True) + bor[0, 0]

    blk = lambda r, cdim: pl.BlockSpec((r, cdim), lambda i: (i, 0))
    full = lambda shape: pl.BlockSpec(shape, lambda i: (0, 0))
    return pl.pallas_call(
        body,
        grid=grid,
        in_specs=[blk(RPT, DC)] * na + [blk(RPT, 1)] + [blk(RPT, DC)] * na
                 + [full((DH, DH)), full((1, DH)), full((DH, DH)),
                    full((1, DH)), full((1, 1))],
        out_specs=blk(RPT, 1),
        out_shape=jax.ShapeDtypeStruct((NP, 1), jnp.float32),
    )(*aggs, deg1, *ys, Wl2, bl2.reshape(1, DH), Wr2, woT, bo)


def kernel(x, edge_index, Wl1, bl1, Wr1, Wl2, bl2, Wr2, Wo, bo):
    src = edge_index[0]
    dst = edge_index[1]
    pad = EP - E
    srcp = jnp.concatenate(
        [src, jnp.full((pad,), N, jnp.int32)]).reshape(NTILES * NH,
                                                       HSTEPS, B)
    dstp = jnp.concatenate(
        [dst, jnp.full((pad,), N, jnp.int32)]).reshape(NTILES * NH,
                                                       HSTEPS, B)

    xp = jnp.zeros((NP, DIN), jnp.float32).at[:N].set(x)
    xchunks = [xp[:, i * DC:(i + 1) * DC] for i in range(DIN // DC)]

    zrows = jnp.zeros((RPT, DC), jnp.float32)
    zdeg = jnp.zeros((RPT,), jnp.float32)
    ones1 = jnp.ones((B,), jnp.float32)

    outs1 = _sc_segsum(xchunks, srcp, dstp, zrows, zdeg, ones1,
                       with_deg=True)
    aggs1, deg = outs1[:-1], outs1[-1]
    deg1 = deg.reshape(NP, 1)
    ys = _tc_layer1(aggs1, deg1, xp, Wl1, bl1, Wr1)
    aggs2 = _sc_segsum(list(ys), srcp, dstp, zrows, zdeg, ones1,
                       with_deg=False)
    out = _tc_layer2(aggs2, deg1, ys, Wl2, bl2, Wr2,
                     Wo.reshape(1, DH), bo.reshape(1, 1))
    return out[:N, 0]
